# Initial kernel scaffold; baseline (speedup 1.0000x reference)
import jax
import jax.numpy as jnp
from jax.experimental import pallas as pl


def kernel(ngrams, weight):
    raise NotImplementedError("write your pallas kernel here")


# speedup vs baseline: 2.9068x; 2.9068x over previous
import functools

import jax
import jax.numpy as jnp
from jax import lax
from jax.experimental import pallas as pl
from jax.experimental.pallas import tpu as pltpu
from jax.experimental.pallas import tpu_sc as plsc

_B = 16384
_L = 50
_D = 32
_NC = 2
_NS = 16
_NW = _NC * _NS
_BPW = _B // _NW
_C = 16
_CR = _C * _L
_NCHUNK = _BPW // _C

_mesh = plsc.VectorSubcoreMesh(core_axis_name="c", subcore_axis_name="s")


@functools.partial(
    pl.kernel,
    mesh=_mesh,
    compiler_params=pltpu.CompilerParams(use_tc_tiling_on_sc=False),
    out_type=jax.ShapeDtypeStruct((_B, _D), jnp.float32),
    scratch_types=[
        pltpu.VMEM((_CR,), jnp.int32),
        pltpu.VMEM((_CR,), jnp.int32),
        pltpu.VMEM((_CR, _D), jnp.float32),
        pltpu.VMEM((_CR, _D), jnp.float32),
        pltpu.VMEM((_BPW, _D), jnp.float32),
        pltpu.SemaphoreType.DMA,
        pltpu.SemaphoreType.DMA,
    ],
)
def _embed_mean(ngrams_hbm, table_hbm, out_hbm,
                idx0, idx1, rows0, rows1, out_v, sem0, sem1):
    wid = lax.axis_index("s") * _NC + lax.axis_index("c")
    ibase = wid * _BPW * _L

    idx_b = (idx0, idx1)
    row_b = (rows0, rows1)
    sem_b = (sem0, sem1)

    def load_chunk(g, buf):
        pltpu.sync_copy(ngrams_hbm.at[pl.ds(ibase + g * _CR, _CR)], idx_b[buf])
        pltpu.async_copy(table_hbm.at[idx_b[buf]], row_b[buf], sem_b[buf])

    def wait_chunk(buf):
        pltpu.make_async_copy(
            table_hbm.at[idx_b[buf]], row_b[buf], sem_b[buf]).wait()

    inv = jnp.float32(1.0 / _L)

    def reduce_chunk(g, buf):
        rows = row_b[buf]

        def bag(i, carry):
            r0 = i * _L
            a0 = rows[r0, pl.ds(0, 16)]
            a1 = rows[r0, pl.ds(16, 16)]
            for j in range(1, _L):
                a0 = a0 + rows[r0 + j, pl.ds(0, 16)]
                a1 = a1 + rows[r0 + j, pl.ds(16, 16)]
            o = g * _C + i
            out_v[o, pl.ds(0, 16)] = a0 * inv
            out_v[o, pl.ds(16, 16)] = a1 * inv
            return carry

        lax.fori_loop(0, _C, bag, 0)

    load_chunk(0, 0)

    def pair(p, carry):
        for half in range(2):
            g = 2 * p + half
            cur = half
            nxt = 1 - half

            @pl.when(g + 1 < _NCHUNK)
            def _():
                load_chunk(g + 1, nxt)

            wait_chunk(cur)
            reduce_chunk(g, cur)
        return carry

    lax.fori_loop(0, _NCHUNK // 2, pair, 0)

    pltpu.sync_copy(out_v, out_hbm.at[pl.ds(wid * _BPW, _BPW)])


def kernel(ngrams, weight):
    return _embed_mean(ngrams.reshape(-1).astype(jnp.int32), weight)
